# Initial kernel scaffold; baseline (speedup 1.0000x reference)
import jax
import jax.numpy as jnp
from jax.experimental import pallas as pl


def kernel(x, edge_index, stroke_idx, batch, Wlh, blh, Wl, bl, Wgh, bgh, Wg, bg, Wp, bp, W1, b1, W2, b2):
    raise NotImplementedError("write your pallas kernel here")


# speedup vs baseline: 18.0637x; 18.0637x over previous
import functools

import jax
import jax.numpy as jnp
from jax import lax
from jax.experimental import pallas as pl
from jax.experimental.pallas import tpu as pltpu
from jax.experimental.pallas import tpu_sc as plsc

_N = 10000
_E = 320000
_F = 128
_C = 64
_NB = 4
_PC = 128
_MLP0 = 256
_OUT = 8
_NSTROKE = 256
_NGRAPH = 16

_NC, _NS, _L = 2, 16, 16
_NW = _NC * _NS
_EPT = _E // _NW
_CH = 128
_NCHUNK = 80
_EPAD = _NCHUNK * _CH
_D = 2 * _C
_AROWS = 10112
_ZPT = _AROWS // _NS
_NPAD = 12288
_NPT = _NPAD // _NW
_PCH = 128
_PROWS = 384
_RPT = _PROWS // _NS

_MESH = plsc.VectorSubcoreMesh(
    core_axis_name="c", subcore_axis_name="s", num_cores=_NC, num_subcores=_NS)


def _wid():
  return lax.axis_index("s") * _NC + lax.axis_index("c")


@functools.partial(
    pl.kernel,
    out_type=jax.ShapeDtypeStruct((_NC * _AROWS,), jnp.float32),
    mesh=_MESH,
    scratch_types=[
        pltpu.VMEM((_NCHUNK, 2, _CH), jnp.int32),
        pltpu.VMEM((_CH,), jnp.float32),
        pltpu.VMEM((640,), jnp.float32),
        pltpu.VMEM_SHARED((_AROWS,), jnp.float32),
    ],
)
def _sc_deg(edgeq, out, eq_v, ones, zeros, acc):
  cid = lax.axis_index("c")
  sid = lax.axis_index("s")
  wid = _wid()
  one = jnp.full((_L,), 1.0, jnp.float32)
  zero = jnp.zeros((_L,), jnp.float32)
  for k in range(_CH // _L):
    ones[pl.ds(k * _L, _L)] = one

  def zfill(r, _):
    zeros[pl.ds(r * _L, _L)] = zero
    return 0
  lax.fori_loop(0, 640 // _L, zfill, 0)
  base = pl.multiple_of(sid * _ZPT, 8)
  pltpu.sync_copy(zeros.at[pl.ds(0, _ZPT)], acc.at[pl.ds(base, _ZPT)])
  pltpu.sync_copy(edgeq.at[wid], eq_v)
  plsc.subcore_barrier()

  def step(j, _):
    pltpu.sync_copy(ones, acc.at[eq_v.at[j, 1]], add=True)
    return 0
  lax.fori_loop(0, _NCHUNK, step, 0)
  plsc.subcore_barrier()
  obase = pl.multiple_of(cid * _AROWS + sid * _ZPT, 8)
  pltpu.sync_copy(acc.at[pl.ds(base, _ZPT)], zeros.at[pl.ds(0, _ZPT)])
  pltpu.sync_copy(zeros.at[pl.ds(0, _ZPT)], out.at[pl.ds(obase, _ZPT)])


@functools.partial(
    pl.kernel,
    out_type=jax.ShapeDtypeStruct((_NC, _AROWS, _D), jnp.float32),
    mesh=_MESH,
    scratch_types=[
        pltpu.VMEM((4, 2, _CH), jnp.int32),
        pltpu.VMEM((2, _CH, _D), jnp.float32),
        pltpu.VMEM_SHARED((_AROWS, _D), jnp.float32),
        [pltpu.SemaphoreType.DMA] * 4,
        [pltpu.SemaphoreType.DMA] * 2,
        [pltpu.SemaphoreType.DMA] * 2,
    ],
)
def _sc_scatter(tbl, edgeq, out, ibuf, gbuf, acc, isem, gsem, ssem):
  cid = lax.axis_index("c")
  sid = lax.axis_index("s")
  wid = _wid()

  for p in range(3):
    pltpu.async_copy(edgeq.at[wid, p], ibuf.at[p], isem[p])
  pltpu.make_async_copy(edgeq.at[wid, 0], ibuf.at[0], isem[0]).wait()
  pltpu.async_copy(tbl.at[ibuf.at[0, 0]], gbuf.at[0], gsem[0])

  zero = jnp.zeros((_L,), jnp.float32)

  def zrow(r, _):
    for c in range(_D // _L):
      gbuf[1, r, pl.ds(c * _L, _L)] = zero
    return 0
  lax.fori_loop(0, _CH, zrow, 0)
  base = pl.multiple_of(sid * _ZPT, 8)
  for k in range(_ZPT // _CH):
    pltpu.sync_copy(gbuf.at[1], acc.at[pl.ds(base + k * _CH, _CH)])
  rem = _ZPT - (_ZPT // _CH) * _CH
  if rem:
    pltpu.sync_copy(gbuf.at[1, pl.ds(0, rem)],
                    acc.at[pl.ds(base + _ZPT - rem, rem)])
  plsc.subcore_barrier()

  def wait_i(p):
    pltpu.make_async_copy(edgeq.at[wid, 0], ibuf.at[p], isem[p]).wait()

  def wait_g(q):
    pltpu.make_async_copy(tbl.at[ibuf.at[0, 0]], gbuf.at[q], gsem[q]).wait()

  def wait_s(q):
    pltpu.make_async_copy(gbuf.at[q], acc.at[ibuf.at[0, 1]], ssem[q]).wait()

  def step4(t, _):
    for k in range(4):
      j = 4 * t + k
      p = k
      q = k % 2
      wait_g(q)

      @pl.when(j >= 1)
      def _():
        wait_s(1 - q)
      @pl.when(j + 3 < _NCHUNK)
      def _():
        pltpu.async_copy(edgeq.at[wid, j + 3], ibuf.at[(k + 3) % 4],
                         isem[(k + 3) % 4])
      @pl.when(j + 1 < _NCHUNK)
      def _():
        wait_i((k + 1) % 4)
        pltpu.async_copy(tbl.at[ibuf.at[(k + 1) % 4, 0]], gbuf.at[1 - q],
                         gsem[1 - q])
      pltpu.async_copy(gbuf.at[q], acc.at[ibuf.at[p, 1]], ssem[q], add=True)
    return 0
  lax.fori_loop(0, _NCHUNK // 4, step4, 0)
  wait_s(1)
  plsc.subcore_barrier()
  pltpu.sync_copy(acc.at[pl.ds(base, _ZPT)], out.at[cid, pl.ds(base, _ZPT)])


@functools.partial(
    pl.kernel,
    out_type=jax.ShapeDtypeStruct((_NW, _PROWS, _D), jnp.float32),
    mesh=_MESH,
    scratch_types=[
        pltpu.VMEM((_NPT, _D), jnp.float32),
        pltpu.VMEM((2, 3, _PCH), jnp.int32),
        pltpu.VMEM((_PROWS, _D), jnp.float32),
    ],
)
def _sc_pool1(hp, sb, out, hpb, idxv, pacc):
  wid = _wid()
  nbase = pl.multiple_of(wid * _NPT, 8)
  pltpu.sync_copy(hp.at[pl.ds(nbase, _NPT)], hpb)
  pltpu.sync_copy(sb.at[wid], idxv)
  neg = jnp.full((_L,), -1e30, jnp.float32)

  def initrow(r, _):
    for c in range(_D // _L):
      pacc[r, pl.ds(c * _L, _L)] = neg
    return 0
  lax.fori_loop(0, _PROWS, initrow, 0)

  def body(g, _):
    k = g // (_PCH // _L)
    r0 = (g % (_PCH // _L)) * _L
    sv = idxv[0, k, pl.ds(r0, _L)]
    bv = idxv[1, k, pl.ds(r0, _L)]
    for l in range(_L):
      i = g * _L + l
      s_id = sv[l]
      b_id = bv[l]
      for c in range(_D // _L):
        sl = pl.ds(c * _L, _L)
        v = hpb[i, sl]
        pacc[s_id, sl] = jnp.maximum(pacc[s_id, sl], v)
        pacc[b_id, sl] = jnp.maximum(pacc[b_id, sl], v)
    return 0
  lax.fori_loop(0, _NPT // _L, body, 0)
  pltpu.sync_copy(pacc, out.at[wid])


@functools.partial(
    pl.kernel,
    out_type=jax.ShapeDtypeStruct((_NPAD, _D), jnp.float32),
    mesh=_MESH,
    scratch_types=[
        pltpu.VMEM((_RPT, _D), jnp.float32),
        pltpu.VMEM((_RPT, _D), jnp.float32),
        pltpu.VMEM((2, 3, _PCH), jnp.int32),
        pltpu.VMEM((_PCH, _D), jnp.float32),
        pltpu.VMEM((_PCH, _D), jnp.float32),
        pltpu.VMEM_SHARED((_PROWS, _D), jnp.float32),
        pltpu.SemaphoreType.DMA,
        pltpu.SemaphoreType.DMA,
    ],
)
def _sc_pool2(part, sb, xg, vacc, pbuf, idxv, bufA, bufB, sfull, semA, semB):
  sid = lax.axis_index("s")
  wid = _wid()
  rbase = pl.multiple_of(sid * _RPT, 8)
  pltpu.sync_copy(part.at[0, pl.ds(rbase, _RPT)], vacc)

  def redw(w, _):
    pltpu.sync_copy(part.at[w, pl.ds(rbase, _RPT)], pbuf)

    def maxrow(r, _):
      for c in range(_D // _L):
        sl = pl.ds(c * _L, _L)
        vacc[r, sl] = jnp.maximum(vacc[r, sl], pbuf[r, sl])
      return 0
    lax.fori_loop(0, _RPT, maxrow, 0)
    return 0
  lax.fori_loop(1, _NW, redw, 0)
  pltpu.sync_copy(vacc, sfull.at[pl.ds(rbase, _RPT)])
  pltpu.sync_copy(sb.at[wid], idxv)
  plsc.subcore_barrier()

  nbase = pl.multiple_of(wid * _NPT, 8)
  for k in range(3):
    pltpu.async_copy(sfull.at[idxv.at[0, k]], bufA, semA)
    pltpu.async_copy(sfull.at[idxv.at[1, k]], bufB, semB)
    pltpu.make_async_copy(sfull.at[idxv.at[0, k]], bufA, semA).wait()
    pltpu.make_async_copy(sfull.at[idxv.at[1, k]], bufB, semB).wait()

    def maxrow(r, _):
      for c in range(_D // _L):
        sl = pl.ds(c * _L, _L)
        bufA[r, sl] = jnp.maximum(bufA[r, sl], bufB[r, sl])
      return 0
    lax.fori_loop(0, _PCH, maxrow, 0)
    pltpu.sync_copy(bufA, xg.at[pl.ds(nbase + k * _PCH, _PCH)])


_R = 1000


def _dot(a, b):
  return jax.lax.dot_general(a, b, (((1,), (0,)), ((), ())),
                             preferred_element_type=jnp.float32)


def _tc_pre_body(x_ref, wd_ref, wb_ref, a_ref, b_ref):
  blk = x_ref[...]
  a_ref[...] = _dot(blk, wd_ref[...])
  b_ref[...] = _dot(blk, wb_ref[...])


def _tc_mid_body(has_res, n_out, a_ref, deg_ref, p_ref, bias_ref, *rest):
  if has_res:
    res_ref = rest[0]
    rest = rest[1:]
  if n_out == 3:
    wd_ref, wb_ref, h_ref, a2_ref, b2_ref = rest
  else:
    h_ref = rest[0]
  agg = p_ref[0] + p_ref[1]
  deg = deg_ref[0, :, 0] + deg_ref[1, :, 0]
  z = deg[:, None] * (a_ref[...] + bias_ref[...]) + agg
  h = jnp.maximum(z, 0.0)
  if has_res:
    h = h + res_ref[...]
  h_ref[...] = h
  if n_out == 3:
    a2_ref[...] = _dot(h, wd_ref[...])
    b2_ref[...] = _dot(h, wb_ref[...])


def _tc_final_body(a_ref, deg_ref, p_ref, bias_ref, res_ref, h0_ref, h1_ref,
                   h2_ref, wp_ref, bp_ref, h_ref, hp_ref):
  agg = p_ref[0] + p_ref[1]
  deg = deg_ref[0, :, 0] + deg_ref[1, :, 0]
  h = jnp.maximum(deg[:, None] * (a_ref[...] + bias_ref[...]) + agg, 0.0)
  h = h + res_ref[...]
  h_ref[...] = h
  acc = bp_ref[...] + _dot(h[:, _C:], wp_ref[4])
  for i, r in enumerate((h0_ref, h1_ref, h2_ref, res_ref)):
    acc = acc + _dot(r[...][:, _C:], wp_ref[i])
  hp_ref[...] = acc


def _tc_mlp_body(h0_ref, h1_ref, h2_ref, h3_ref, h4_ref, xg_ref, w1l_ref,
                 w1g_ref, b1_ref, w2_ref, b2_ref, o_ref):
  z = b1_ref[...] + _dot(xg_ref[...], w1g_ref[...])
  for i, r in enumerate((h0_ref, h1_ref, h2_ref, h3_ref, h4_ref)):
    z = z + _dot(r[...][:, :_C], w1l_ref[i])
  z = jnp.maximum(z, 0.0)
  o = b2_ref[...] + _dot(z, w2_ref[...])
  m = jnp.max(o, axis=1, keepdims=True)
  e = jnp.exp(o - m)
  o_ref[...] = (o - m) - jnp.log(jnp.sum(e, axis=1, keepdims=True))


def _row_spec(shape_tail):
  return pl.BlockSpec((_R,) + shape_tail, lambda i: (i,) + (0,) * len(shape_tail))


def _full_spec(shape):
  rank = len(shape)
  return pl.BlockSpec(shape, lambda i: (0,) * rank)


_DEG_SPEC = pl.BlockSpec((_NC, _R, 1), lambda i: (0, i, 0))
_P_SPEC = pl.BlockSpec((_NC, _R, _D), lambda i: (0, i, 0))
_ROW = _row_spec((_D,))


def _tc_pre(x, wd, wb):
  return pl.pallas_call(
      _tc_pre_body,
      grid=(_N // _R,),
      in_specs=[_ROW, _full_spec((_F, _D)), _full_spec((_F, _D))],
      out_specs=[_ROW, _ROW],
      out_shape=[jax.ShapeDtypeStruct((_N, _D), jnp.float32)] * 2,
  )(x, wd, wb)


def _tc_mid(a, deg, p, bias, res, wd, wb):
  has_res = res is not None
  in_specs = [_ROW, _DEG_SPEC, _P_SPEC, _full_spec((1, _D))]
  args = [a, deg, p, bias]
  if has_res:
    in_specs.append(_ROW)
    args.append(res)
  in_specs += [_full_spec((_D, _D)), _full_spec((_D, _D))]
  args += [wd, wb]
  return pl.pallas_call(
      functools.partial(_tc_mid_body, has_res, 3),
      grid=(_N // _R,),
      in_specs=in_specs,
      out_specs=[_ROW, _ROW, _ROW],
      out_shape=[jax.ShapeDtypeStruct((_N, _D), jnp.float32)] * 3,
  )(*args)


def _tc_final(a, deg, p, bias, res, h0, h1, h2, wp, bp):
  return pl.pallas_call(
      _tc_final_body,
      grid=(_N // _R,),
      in_specs=[_ROW, _DEG_SPEC, _P_SPEC, _full_spec((1, _D)), _ROW, _ROW,
                _ROW, _ROW, _full_spec((5, _C, _PC)), _full_spec((1, _PC))],
      out_specs=[_ROW, _ROW],
      out_shape=[jax.ShapeDtypeStruct((_N, _D), jnp.float32),
                 jax.ShapeDtypeStruct((_NPAD, _PC), jnp.float32)],
  )(a, deg, p, bias, res, h0, h1, h2, wp, bp)


def _tc_mlp(hs, xg, w1l, w1g, b1, w2, b2):
  return pl.pallas_call(
      _tc_mlp_body,
      grid=(_N // _R,),
      in_specs=[_ROW] * 5 + [_row_spec((_PC,)), _full_spec((5, _C, _MLP0)),
                             _full_spec((_PC, _MLP0)), _full_spec((1, _MLP0)),
                             _full_spec((_MLP0, _OUT)), _full_spec((1, _OUT))],
      out_specs=_row_spec((_OUT,)),
      out_shape=jax.ShapeDtypeStruct((_N, _OUT), jnp.float32),
  )(*hs, xg, w1l, w1g, b1, w2, b2)


def kernel(x, edge_index, stroke_idx, batch, Wlh, blh, Wl, bl, Wgh, bgh, Wg,
           bg, Wp, bp, W1, b1, W2, b2):
  f32 = jnp.float32
  i32 = jnp.int32
  src = edge_index[0].astype(i32)
  dst = edge_index[1].astype(i32)

  npad_e = _EPAD - _EPT
  pad_src = jnp.broadcast_to(
      (jnp.arange(_NW, dtype=i32) * 131 % _N)[:, None], (_NW, npad_e))
  pad_dst = jnp.broadcast_to(
      (_N + jnp.arange(_NW, dtype=i32) % _L)[:, None], (_NW, npad_e))
  srcq = jnp.concatenate([src.reshape(_NW, _EPT), pad_src], 1)
  dstq = jnp.concatenate([dst.reshape(_NW, _EPT), pad_dst], 1)
  edgeq = jnp.stack([srcq.reshape(_NW, _NCHUNK, _CH),
                     dstq.reshape(_NW, _NCHUNK, _CH)], axis=2)

  pad_n = _NPAD - _N
  pad_pool = (_NSTROKE + _NGRAPH + jnp.arange(pad_n, dtype=i32) % _L)
  sidx = jnp.concatenate([stroke_idx.astype(i32), pad_pool])
  bidx = jnp.concatenate([batch.astype(i32) + _NSTROKE, pad_pool])
  sb = jnp.stack([sidx, bidx], 0).reshape(2, _NW, 3, _PCH).transpose(1, 0, 2, 3)

  def split2(w):
    k = w.shape[0] // 2
    return w[:k], w[k:]

  lt, lb_ = split2(Wlh)
  gt, gb_ = split2(Wgh)
  wds = [jnp.concatenate([lt - lb_, gt - gb_], 1)]
  wbs = [jnp.concatenate([lb_, gb_], 1)]
  biases = [jnp.concatenate([blh, bgh]).reshape(1, _D)]
  z64 = jnp.zeros((_C, _C), f32)
  for i in range(_NB):
    lt, lb_ = split2(Wl[i])
    gt, gb_ = split2(Wg[i])
    wds.append(jnp.concatenate([
        jnp.concatenate([lt - lb_, z64], 1),
        jnp.concatenate([z64, gt - gb_], 1)], 0))
    wbs.append(jnp.concatenate([
        jnp.concatenate([lb_, z64], 1),
        jnp.concatenate([z64, gb_], 1)], 0))
    biases.append(jnp.concatenate([bl[i], bg[i]]).reshape(1, _D))

  wp_s = Wp.reshape(_NB + 1, _C, _PC)
  w1l = W1[:(_NB + 1) * _C].reshape(_NB + 1, _C, _MLP0)
  w1g = W1[(_NB + 1) * _C:]
  b1r = b1.reshape(1, _MLP0)
  bpr = bp.reshape(1, _PC)
  w2r = W2
  b2r = b2.reshape(1, _OUT)

  deg = _sc_deg(edgeq).reshape(_NC, _AROWS, 1)

  a, b = _tc_pre(x, wds[0], wbs[0])
  p = _sc_scatter(b, edgeq)
  h0, a, b = _tc_mid(a, deg, p, biases[0], None, wds[1], wbs[1])
  p = _sc_scatter(b, edgeq)
  h1, a, b = _tc_mid(a, deg, p, biases[1], h0, wds[2], wbs[2])
  p = _sc_scatter(b, edgeq)
  h2, a, b = _tc_mid(a, deg, p, biases[2], h1, wds[3], wbs[3])
  p = _sc_scatter(b, edgeq)
  h3, a, b = _tc_mid(a, deg, p, biases[3], h2, wds[4], wbs[4])
  p = _sc_scatter(b, edgeq)
  h4, hp = _tc_final(a, deg, p, biases[4], h3, h0, h1, h2, wp_s, bpr)

  part = _sc_pool1(hp, sb)
  xg = _sc_pool2(part, sb)
  return _tc_mlp([h0, h1, h2, h3, h4], xg, w1l, w1g, b1r, w2r, b2r)


# speedup vs baseline: 18.9102x; 1.0469x over previous
import functools

import jax
import jax.numpy as jnp
from jax import lax
from jax.experimental import pallas as pl
from jax.experimental.pallas import tpu as pltpu
from jax.experimental.pallas import tpu_sc as plsc

_N = 10000
_E = 320000
_F = 128
_C = 64
_NB = 4
_PC = 128
_MLP0 = 256
_OUT = 8
_NSTROKE = 256
_NGRAPH = 16

_NC, _NS, _L = 2, 16, 16
_NW = _NC * _NS
_EPT = _E // _NW
_CH = 64
_NCHUNK = 160
_EPAD = _NCHUNK * _CH
_D = 2 * _C
_AROWS = 10112
_ZPT = _AROWS // _NS
_NPAD = 12288
_NPT = _NPAD // _NW
_PCH = 128
_PROWS = 384
_RPT = _PROWS // _NS

_MESH = plsc.VectorSubcoreMesh(
    core_axis_name="c", subcore_axis_name="s", num_cores=_NC, num_subcores=_NS)


def _wid():
  return lax.axis_index("s") * _NC + lax.axis_index("c")


@functools.partial(
    pl.kernel,
    out_type=jax.ShapeDtypeStruct((_NC * _AROWS,), jnp.float32),
    mesh=_MESH,
    scratch_types=[
        pltpu.VMEM((_NCHUNK, _CH), jnp.int32),
        pltpu.VMEM((_CH,), jnp.float32),
        pltpu.VMEM((640,), jnp.float32),
        pltpu.VMEM_SHARED((_AROWS,), jnp.float32),
        [pltpu.SemaphoreType.DMA] * 4,
    ],
)
def _sc_deg(dstq, out, idx_d, ones, zeros, acc, sems):
  cid = lax.axis_index("c")
  sid = lax.axis_index("s")
  wid = _wid()
  one = jnp.full((_L,), 1.0, jnp.float32)
  zero = jnp.zeros((_L,), jnp.float32)
  for k in range(_CH // _L):
    ones[pl.ds(k * _L, _L)] = one

  def zfill(r, _):
    zeros[pl.ds(r * _L, _L)] = zero
    return 0
  lax.fori_loop(0, 640 // _L, zfill, 0)
  base = pl.multiple_of(sid * _ZPT, 8)
  pltpu.sync_copy(zeros.at[pl.ds(0, _ZPT)], acc.at[pl.ds(base, _ZPT)])
  pltpu.sync_copy(dstq.at[wid], idx_d)
  plsc.subcore_barrier()

  def step(t, _):
    for k in range(4):
      j = 4 * t + k

      @pl.when(j >= 4)
      def _():
        pltpu.make_async_copy(ones, acc.at[idx_d.at[0]], sems[k]).wait()
      pltpu.async_copy(ones, acc.at[idx_d.at[j]], sems[k], add=True)
    return 0
  lax.fori_loop(0, _NCHUNK // 4, step, 0)
  for k in range(4):
    pltpu.make_async_copy(ones, acc.at[idx_d.at[0]], sems[k]).wait()
  plsc.subcore_barrier()
  obase = pl.multiple_of(cid * _AROWS + sid * _ZPT, 8)
  pltpu.sync_copy(acc.at[pl.ds(base, _ZPT)], zeros.at[pl.ds(0, _ZPT)])
  pltpu.sync_copy(zeros.at[pl.ds(0, _ZPT)], out.at[pl.ds(obase, _ZPT)])


@functools.partial(
    pl.kernel,
    out_type=jax.ShapeDtypeStruct((_NC, _AROWS, _D), jnp.float32),
    mesh=_MESH,
    scratch_types=[
        pltpu.VMEM((8, 2, _CH), jnp.int32),
        pltpu.VMEM((4, _CH, _D), jnp.float32),
        pltpu.VMEM_SHARED((_AROWS, _D), jnp.float32),
        [pltpu.SemaphoreType.DMA] * 8,
        [pltpu.SemaphoreType.DMA] * 4,
        [pltpu.SemaphoreType.DMA] * 4,
    ],
)
def _sc_scatter(tbl, edgeq, out, ibuf, gbuf, acc, isem, gsem, ssem):
  cid = lax.axis_index("c")
  sid = lax.axis_index("s")
  wid = _wid()

  def wait_i(p):
    pltpu.make_async_copy(edgeq.at[wid, 0], ibuf.at[p], isem[p]).wait()

  def wait_g(q):
    pltpu.make_async_copy(tbl.at[ibuf.at[0, 0]], gbuf.at[q], gsem[q]).wait()

  def wait_s(q):
    pltpu.make_async_copy(gbuf.at[q], acc.at[ibuf.at[0, 1]], ssem[q]).wait()

  for p in range(5):
    pltpu.async_copy(edgeq.at[wid, p], ibuf.at[p], isem[p])
  wait_i(0)
  pltpu.async_copy(tbl.at[ibuf.at[0, 0]], gbuf.at[0], gsem[0])
  wait_i(1)
  pltpu.async_copy(tbl.at[ibuf.at[1, 0]], gbuf.at[1], gsem[1])

  zero = jnp.zeros((_L,), jnp.float32)

  def zrow(r, _):
    for c in range(_D // _L):
      gbuf[3, r, pl.ds(c * _L, _L)] = zero
    return 0
  lax.fori_loop(0, _CH, zrow, 0)
  base = pl.multiple_of(sid * _ZPT, 8)
  for k in range(_ZPT // _CH):
    pltpu.sync_copy(gbuf.at[3], acc.at[pl.ds(base + k * _CH, _CH)])
  rem = _ZPT - (_ZPT // _CH) * _CH
  if rem:
    pltpu.sync_copy(gbuf.at[3, pl.ds(0, rem)],
                    acc.at[pl.ds(base + _ZPT - rem, rem)])
  plsc.subcore_barrier()

  def step8(t, _):
    for k in range(8):
      j = 8 * t + k
      q = k % 4
      wait_g(q)

      @pl.when(j >= 2)
      def _():
        wait_s((k + 2) % 4)

      @pl.when(j + 5 < _NCHUNK)
      def _():
        pltpu.async_copy(edgeq.at[wid, j + 5], ibuf.at[(k + 5) % 8],
                         isem[(k + 5) % 8])

      @pl.when(j + 2 < _NCHUNK)
      def _():
        wait_i((k + 2) % 8)
        pltpu.async_copy(tbl.at[ibuf.at[(k + 2) % 8, 0]], gbuf.at[(k + 2) % 4],
                         gsem[(k + 2) % 4])
      pltpu.async_copy(gbuf.at[q], acc.at[ibuf.at[k, 1]], ssem[q], add=True)
    return 0
  lax.fori_loop(0, _NCHUNK // 8, step8, 0)
  wait_s((_NCHUNK - 2) % 4)
  wait_s((_NCHUNK - 1) % 4)
  plsc.subcore_barrier()
  pltpu.sync_copy(acc.at[pl.ds(base, _ZPT)], out.at[cid, pl.ds(base, _ZPT)])


@functools.partial(
    pl.kernel,
    out_type=jax.ShapeDtypeStruct((_NW, _PROWS, _D), jnp.float32),
    mesh=_MESH,
    scratch_types=[
        pltpu.VMEM((_NPT, _D), jnp.float32),
        pltpu.VMEM((2, 3, _PCH), jnp.int32),
        pltpu.VMEM((_PROWS, _D), jnp.float32),
    ],
)
def _sc_pool1(hp, sb, out, hpb, idxv, pacc):
  wid = _wid()
  nbase = pl.multiple_of(wid * _NPT, 8)
  pltpu.sync_copy(hp.at[pl.ds(nbase, _NPT)], hpb)
  pltpu.sync_copy(sb.at[wid], idxv)
  neg = jnp.full((_L,), -1e30, jnp.float32)

  def initrow(r, _):
    for c in range(_D // _L):
      pacc[r, pl.ds(c * _L, _L)] = neg
    return 0
  lax.fori_loop(0, _PROWS, initrow, 0)

  def body(g, _):
    k = g // (_PCH // _L)
    r0 = (g % (_PCH // _L)) * _L
    sv = idxv[0, k, pl.ds(r0, _L)]
    bv = idxv[1, k, pl.ds(r0, _L)]
    for l in range(_L):
      i = g * _L + l
      s_id = sv[l]
      b_id = bv[l]
      for c in range(_D // _L):
        sl = pl.ds(c * _L, _L)
        v = hpb[i, sl]
        pacc[s_id, sl] = jnp.maximum(pacc[s_id, sl], v)
        pacc[b_id, sl] = jnp.maximum(pacc[b_id, sl], v)
    return 0
  lax.fori_loop(0, _NPT // _L, body, 0)
  pltpu.sync_copy(pacc, out.at[wid])


@functools.partial(
    pl.kernel,
    out_type=jax.ShapeDtypeStruct((_NPAD, _D), jnp.float32),
    mesh=_MESH,
    scratch_types=[
        pltpu.VMEM((_RPT, _D), jnp.float32),
        pltpu.VMEM((_RPT, _D), jnp.float32),
        pltpu.VMEM((2, 3, _PCH), jnp.int32),
        pltpu.VMEM((_PCH, _D), jnp.float32),
        pltpu.VMEM((_PCH, _D), jnp.float32),
        pltpu.VMEM_SHARED((_PROWS, _D), jnp.float32),
        pltpu.SemaphoreType.DMA,
        pltpu.SemaphoreType.DMA,
    ],
)
def _sc_pool2(part, sb, xg, vacc, pbuf, idxv, bufA, bufB, sfull, semA, semB):
  sid = lax.axis_index("s")
  wid = _wid()
  rbase = pl.multiple_of(sid * _RPT, 8)
  pltpu.sync_copy(part.at[0, pl.ds(rbase, _RPT)], vacc)

  def redw(w, _):
    pltpu.sync_copy(part.at[w, pl.ds(rbase, _RPT)], pbuf)

    def maxrow(r, _):
      for c in range(_D // _L):
        sl = pl.ds(c * _L, _L)
        vacc[r, sl] = jnp.maximum(vacc[r, sl], pbuf[r, sl])
      return 0
    lax.fori_loop(0, _RPT, maxrow, 0)
    return 0
  lax.fori_loop(1, _NW, redw, 0)
  pltpu.sync_copy(vacc, sfull.at[pl.ds(rbase, _RPT)])
  pltpu.sync_copy(sb.at[wid], idxv)
  plsc.subcore_barrier()

  nbase = pl.multiple_of(wid * _NPT, 8)
  for k in range(3):
    pltpu.async_copy(sfull.at[idxv.at[0, k]], bufA, semA)
    pltpu.async_copy(sfull.at[idxv.at[1, k]], bufB, semB)
    pltpu.make_async_copy(sfull.at[idxv.at[0, k]], bufA, semA).wait()
    pltpu.make_async_copy(sfull.at[idxv.at[1, k]], bufB, semB).wait()

    def maxrow(r, _):
      for c in range(_D // _L):
        sl = pl.ds(c * _L, _L)
        bufA[r, sl] = jnp.maximum(bufA[r, sl], bufB[r, sl])
      return 0
    lax.fori_loop(0, _PCH, maxrow, 0)
    pltpu.sync_copy(bufA, xg.at[pl.ds(nbase + k * _PCH, _PCH)])


_R = 1000


def _dot(a, b):
  return jax.lax.dot_general(a, b, (((1,), (0,)), ((), ())),
                             preferred_element_type=jnp.float32)


def _tc_pre_body(x_ref, wd_ref, wb_ref, a_ref, b_ref):
  blk = x_ref[...]
  a_ref[...] = _dot(blk, wd_ref[...])
  b_ref[...] = _dot(blk, wb_ref[...])


def _tc_mid_body(has_res, n_out, a_ref, deg_ref, p_ref, bias_ref, *rest):
  if has_res:
    res_ref = rest[0]
    rest = rest[1:]
  if n_out == 3:
    wd_ref, wb_ref, h_ref, a2_ref, b2_ref = rest
  else:
    h_ref = rest[0]
  agg = p_ref[0] + p_ref[1]
  deg = deg_ref[0, :, 0] + deg_ref[1, :, 0]
  z = deg[:, None] * (a_ref[...] + bias_ref[...]) + agg
  h = jnp.maximum(z, 0.0)
  if has_res:
    h = h + res_ref[...]
  h_ref[...] = h
  if n_out == 3:
    a2_ref[...] = _dot(h, wd_ref[...])
    b2_ref[...] = _dot(h, wb_ref[...])


def _tc_final_body(a_ref, deg_ref, p_ref, bias_ref, res_ref, h0_ref, h1_ref,
                   h2_ref, wp_ref, bp_ref, h_ref, hp_ref):
  agg = p_ref[0] + p_ref[1]
  deg = deg_ref[0, :, 0] + deg_ref[1, :, 0]
  h = jnp.maximum(deg[:, None] * (a_ref[...] + bias_ref[...]) + agg, 0.0)
  h = h + res_ref[...]
  h_ref[...] = h
  acc = bp_ref[...] + _dot(h[:, _C:], wp_ref[4])
  for i, r in enumerate((h0_ref, h1_ref, h2_ref, res_ref)):
    acc = acc + _dot(r[...][:, _C:], wp_ref[i])
  hp_ref[...] = acc


def _tc_mlp_body(h0_ref, h1_ref, h2_ref, h3_ref, h4_ref, xg_ref, w1l_ref,
                 w1g_ref, b1_ref, w2_ref, b2_ref, o_ref):
  z = b1_ref[...] + _dot(xg_ref[...], w1g_ref[...])
  for i, r in enumerate((h0_ref, h1_ref, h2_ref, h3_ref, h4_ref)):
    z = z + _dot(r[...][:, :_C], w1l_ref[i])
  z = jnp.maximum(z, 0.0)
  o = b2_ref[...] + _dot(z, w2_ref[...])
  m = jnp.max(o, axis=1, keepdims=True)
  e = jnp.exp(o - m)
  o_ref[...] = (o - m) - jnp.log(jnp.sum(e, axis=1, keepdims=True))


def _row_spec(shape_tail):
  return pl.BlockSpec((_R,) + shape_tail, lambda i: (i,) + (0,) * len(shape_tail))


def _full_spec(shape):
  rank = len(shape)
  return pl.BlockSpec(shape, lambda i: (0,) * rank)


_DEG_SPEC = pl.BlockSpec((_NC, _R, 1), lambda i: (0, i, 0))
_P_SPEC = pl.BlockSpec((_NC, _R, _D), lambda i: (0, i, 0))
_ROW = _row_spec((_D,))


def _tc_pre(x, wd, wb):
  return pl.pallas_call(
      _tc_pre_body,
      grid=(_N // _R,),
      in_specs=[_ROW, _full_spec((_F, _D)), _full_spec((_F, _D))],
      out_specs=[_ROW, _ROW],
      out_shape=[jax.ShapeDtypeStruct((_N, _D), jnp.float32)] * 2,
  )(x, wd, wb)


def _tc_mid(a, deg, p, bias, res, wd, wb):
  has_res = res is not None
  in_specs = [_ROW, _DEG_SPEC, _P_SPEC, _full_spec((1, _D))]
  args = [a, deg, p, bias]
  if has_res:
    in_specs.append(_ROW)
    args.append(res)
  in_specs += [_full_spec((_D, _D)), _full_spec((_D, _D))]
  args += [wd, wb]
  return pl.pallas_call(
      functools.partial(_tc_mid_body, has_res, 3),
      grid=(_N // _R,),
      in_specs=in_specs,
      out_specs=[_ROW, _ROW, _ROW],
      out_shape=[jax.ShapeDtypeStruct((_N, _D), jnp.float32)] * 3,
  )(*args)


def _tc_final(a, deg, p, bias, res, h0, h1, h2, wp, bp):
  return pl.pallas_call(
      _tc_final_body,
      grid=(_N // _R,),
      in_specs=[_ROW, _DEG_SPEC, _P_SPEC, _full_spec((1, _D)), _ROW, _ROW,
                _ROW, _ROW, _full_spec((5, _C, _PC)), _full_spec((1, _PC))],
      out_specs=[_ROW, _ROW],
      out_shape=[jax.ShapeDtypeStruct((_N, _D), jnp.float32),
                 jax.ShapeDtypeStruct((_NPAD, _PC), jnp.float32)],
  )(a, deg, p, bias, res, h0, h1, h2, wp, bp)


def _tc_mlp(hs, xg, w1l, w1g, b1, w2, b2):
  return pl.pallas_call(
      _tc_mlp_body,
      grid=(_N // _R,),
      in_specs=[_ROW] * 5 + [_row_spec((_PC,)), _full_spec((5, _C, _MLP0)),
                             _full_spec((_PC, _MLP0)), _full_spec((1, _MLP0)),
                             _full_spec((_MLP0, _OUT)), _full_spec((1, _OUT))],
      out_specs=_row_spec((_OUT,)),
      out_shape=jax.ShapeDtypeStruct((_N, _OUT), jnp.float32),
  )(*hs, xg, w1l, w1g, b1, w2, b2)


def kernel(x, edge_index, stroke_idx, batch, Wlh, blh, Wl, bl, Wgh, bgh, Wg,
           bg, Wp, bp, W1, b1, W2, b2):
  f32 = jnp.float32
  i32 = jnp.int32
  src = edge_index[0].astype(i32)
  dst = edge_index[1].astype(i32)

  npad_e = _EPAD - _EPT
  pad_src = jnp.broadcast_to(
      (jnp.arange(_NW, dtype=i32) * 131 % _N)[:, None], (_NW, npad_e))
  pad_dst = jnp.broadcast_to(
      (_N + jnp.arange(_NW, dtype=i32) % _L)[:, None], (_NW, npad_e))
  srcq = jnp.concatenate([src.reshape(_NW, _EPT), pad_src], 1)
  dstq = jnp.concatenate([dst.reshape(_NW, _EPT), pad_dst], 1)
  dstq_c = dstq.reshape(_NW, _NCHUNK, _CH)
  edgeq = jnp.stack([srcq.reshape(_NW, _NCHUNK, _CH), dstq_c], axis=2)

  pad_n = _NPAD - _N
  pad_pool = (_NSTROKE + _NGRAPH + jnp.arange(pad_n, dtype=i32) % _L)
  sidx = jnp.concatenate([stroke_idx.astype(i32), pad_pool])
  bidx = jnp.concatenate([batch.astype(i32) + _NSTROKE, pad_pool])
  sb = jnp.stack([sidx, bidx], 0).reshape(2, _NW, 3, _PCH).transpose(1, 0, 2, 3)

  def split2(w):
    k = w.shape[0] // 2
    return w[:k], w[k:]

  lt, lb_ = split2(Wlh)
  gt, gb_ = split2(Wgh)
  wds = [jnp.concatenate([lt - lb_, gt - gb_], 1)]
  wbs = [jnp.concatenate([lb_, gb_], 1)]
  biases = [jnp.concatenate([blh, bgh]).reshape(1, _D)]
  z64 = jnp.zeros((_C, _C), f32)
  for i in range(_NB):
    lt, lb_ = split2(Wl[i])
    gt, gb_ = split2(Wg[i])
    wds.append(jnp.concatenate([
        jnp.concatenate([lt - lb_, z64], 1),
        jnp.concatenate([z64, gt - gb_], 1)], 0))
    wbs.append(jnp.concatenate([
        jnp.concatenate([lb_, z64], 1),
        jnp.concatenate([z64, gb_], 1)], 0))
    biases.append(jnp.concatenate([bl[i], bg[i]]).reshape(1, _D))

  wp_s = Wp.reshape(_NB + 1, _C, _PC)
  w1l = W1[:(_NB + 1) * _C].reshape(_NB + 1, _C, _MLP0)
  w1g = W1[(_NB + 1) * _C:]
  b1r = b1.reshape(1, _MLP0)
  bpr = bp.reshape(1, _PC)
  w2r = W2
  b2r = b2.reshape(1, _OUT)

  deg = _sc_deg(dstq_c).reshape(_NC, _AROWS, 1)

  a, b = _tc_pre(x, wds[0], wbs[0])
  p = _sc_scatter(b, edgeq)
  h0, a, b = _tc_mid(a, deg, p, biases[0], None, wds[1], wbs[1])
  p = _sc_scatter(b, edgeq)
  h1, a, b = _tc_mid(a, deg, p, biases[1], h0, wds[2], wbs[2])
  p = _sc_scatter(b, edgeq)
  h2, a, b = _tc_mid(a, deg, p, biases[2], h1, wds[3], wbs[3])
  p = _sc_scatter(b, edgeq)
  h3, a, b = _tc_mid(a, deg, p, biases[3], h2, wds[4], wbs[4])
  p = _sc_scatter(b, edgeq)
  h4, hp = _tc_final(a, deg, p, biases[4], h3, h0, h1, h2, wp_s, bpr)

  part = _sc_pool1(hp, sb)
  xg = _sc_pool2(part, sb)
  return _tc_mlp([h0, h1, h2, h3, h4], xg, w1l, w1g, b1r, w2r, b2r)
